# baseline (speedup 1.0000x reference)
import math

import jax
import jax.numpy as jnp
import numpy as np
from jax.experimental import pallas as pl
from jax.experimental.pallas import tpu as pltpu

N = 10000
E = 160000
T = 480000
B = 256
H = 128
NR = 6
NS = 7
INT_EMB = 64
BAS = 8
OUT_EMB = 256
OUT_C = 16
NB = 4
CUTOFF = 5.0

SQ2C = math.sqrt(2.0 / CUTOFF)
EBLK = 2000
NBLK = 2000
TBLK = 4000


def _sig(x):
    return 1.0 / (1.0 + jnp.exp(-x))


def _swish(x):
    return x * _sig(x)


def _dotf(a, b):
    return jax.lax.dot_general(a, b, (((1,), (0,)), ((), ())),
                               precision=jax.lax.Precision.HIGHEST,
                               preferred_element_type=jnp.float32)


def _sel(rows, cols, pairs):
    m = np.zeros((rows, cols), np.float32)
    for r, c in pairs:
        m[r, c] = 1.0
    return jnp.asarray(m)


def _edge_init_body(erec0_ref, sels_ref, selz_ref, fr_ref,
                    tw1_ref, tw2_ref, w3r_ref, b3r_ref, w3_ref,
                    blin_ref, r0_ref, erec_ref, x_ref, m0_ref):
    e0 = erec0_ref[...]
    ss16 = _dotf(e0, sels_ref[0])
    dist16 = jnp.sqrt(ss16 + 1e-12)
    inv16 = 1.0 / dist16
    freq_a = fr_ref[0:1]
    freq_b = fr_ref[1:2]
    mask3 = fr_ref[2:3]
    rbf16 = SQ2C * jnp.sin(dist16 * freq_a) * inv16
    rad16 = jnp.sin(dist16 * freq_b) * inv16
    xyz = _dotf(e0, sels_ref[1])
    erec_ref[...] = xyz + dist16 * mask3 + rad16

    zj96 = _dotf(e0, selz_ref[0])
    zi96 = _dotf(e0, selz_ref[1])
    iota96 = jax.lax.broadcasted_iota(jnp.int32, (1, 96), 1).astype(jnp.float32)
    ohj = (zj96 == iota96).astype(jnp.float32)
    ohi = (zi96 == iota96).astype(jnp.float32)
    rbf_e = _swish(_dotf(rbf16, w3r_ref[...]) + b3r_ref[...])
    x = _swish(_dotf(ohj, tw1_ref[...]) + _dotf(ohi, tw2_ref[...]) +
               _dotf(rbf_e, w3_ref[...]) + blin_ref[...])
    x_ref[...] = x
    m0_ref[...] = x * _dotf(rbf16, r0_ref[...])


def _edge_init(erec0, tw1, tw2, w3r, b3r, w3, blin, r0):
    sels = jnp.stack([_sel(8, 16, [(3, c) for c in range(16)]),
                      _sel(8, 16, [(0, 0), (1, 1), (2, 2)])])
    selz = jnp.stack([_sel(8, 96, [(4, c) for c in range(96)]),
                      _sel(8, 96, [(5, c) for c in range(96)])])
    n_pi_c = np.array([(n + 1) * math.pi / CUTOFF for n in range(NR)],
                      np.float32)
    fr = np.zeros((3, 16), np.float32)
    fr[0, 0:NR] = n_pi_c
    fr[1, 4:4 + NR] = n_pi_c
    fr[2, 3] = 1.0
    full = lambda s: pl.BlockSpec(s, lambda *a: tuple(0 for _ in s))
    return pl.pallas_call(
        _edge_init_body,
        grid=(E // EBLK,),
        in_specs=[pl.BlockSpec((EBLK, 8), lambda i: (i, 0)),
                  full((2, 8, 16)), full((2, 8, 96)), full((3, 16)),
                  full((96, H)), full((96, H)), full((16, H)), full((1, H)),
                  full((H, H)), full((1, H)), full((16, H))],
        out_specs=[pl.BlockSpec((EBLK, 16), lambda i: (i, 0)),
                   pl.BlockSpec((EBLK, H), lambda i: (i, 0)),
                   pl.BlockSpec((EBLK, H), lambda i: (i, 0))],
        out_shape=[jax.ShapeDtypeStruct((E, 16), jnp.float32),
                   jax.ShapeDtypeStruct((E, H), jnp.float32),
                   jax.ShapeDtypeStruct((E, H), jnp.float32)],
    )(erec0, sels, selz, jnp.asarray(fr), tw1, tw2, w3r, b3r, w3, blin, r0)


def _edge_in_body(x_ref, erec_ref, jiw_ref, jib_ref, kjw_ref, kjb_ref,
                  wrbf_ref, down_ref, xji_ref, xkj_ref):
    x = x_ref[...]
    xji_ref[...] = _swish(_dotf(x, jiw_ref[...]) + jib_ref[...])
    t = _swish(_dotf(x, kjw_ref[...]) + kjb_ref[...])
    t = t * _dotf(erec_ref[...], wrbf_ref[...])
    xkj_ref[...] = _swish(_dotf(t, down_ref[...]))


def _edge_in(x, erec, jiw, jib, kjw, kjb, wrbf, down):
    full = lambda s: pl.BlockSpec(s, lambda i: (0, 0))
    return pl.pallas_call(
        _edge_in_body,
        grid=(E // EBLK,),
        in_specs=[pl.BlockSpec((EBLK, H), lambda i: (i, 0)),
                  pl.BlockSpec((EBLK, 16), lambda i: (i, 0)),
                  full((H, H)), full((1, H)), full((H, H)), full((1, H)),
                  full((16, H)), full((H, INT_EMB))],
        out_specs=[pl.BlockSpec((EBLK, H), lambda i: (i, 0)),
                   pl.BlockSpec((EBLK, INT_EMB), lambda i: (i, 0))],
        out_shape=[jax.ShapeDtypeStruct((E, H), jnp.float32),
                   jax.ShapeDtypeStruct((E, INT_EMB), jnp.float32)],
    )(x, erec, jiw, jib, kjw, kjb, wrbf, down)


def _sbf_emb_body(ar_ref, wpad_ref, bsel_ref, semb_ref):
    ar = ar_ref[...]
    acc = jnp.zeros((ar.shape[0], INT_EMB), jnp.float32)
    for l in range(NS):
        acc = acc + _dotf(ar, bsel_ref[l]) * _dotf(ar, wpad_ref[l])
    semb_ref[...] = acc


def _sbf_emb(angrad, wpad):
    bsel = jnp.stack([_sel(16, INT_EMB, [(l, c) for c in range(INT_EMB)])
                      for l in range(NS)])
    return pl.pallas_call(
        _sbf_emb_body,
        grid=(T // TBLK,),
        in_specs=[pl.BlockSpec((TBLK, 16), lambda i: (i, 0)),
                  pl.BlockSpec((NS, 16, INT_EMB), lambda i: (0, 0, 0)),
                  pl.BlockSpec((NS, 16, INT_EMB), lambda i: (0, 0, 0))],
        out_specs=pl.BlockSpec((TBLK, INT_EMB), lambda i: (i, 0)),
        out_shape=jax.ShapeDtypeStruct((T, INT_EMB), jnp.float32),
    )(angrad, wpad, bsel)


def _edge_out_body(agg_ref, xji_ref, x_ref, erec_ref, up_ref,
                   r1aw_ref, r1ab_ref, r1bw_ref, r1bb_ref,
                   linw_ref, linb_ref,
                   r2aw_ref, r2ab_ref, r2bw_ref, r2bb_ref,
                   rpad_ref, xn_ref, m_ref):
    h = xji_ref[...] + _swish(_dotf(agg_ref[...], up_ref[...]))
    h = h + _swish(_dotf(_swish(_dotf(h, r1aw_ref[...]) + r1ab_ref[...]),
                         r1bw_ref[...]) + r1bb_ref[...])
    h = _swish(_dotf(h, linw_ref[...]) + linb_ref[...]) + x_ref[...]
    h = h + _swish(_dotf(_swish(_dotf(h, r2aw_ref[...]) + r2ab_ref[...]),
                         r2bw_ref[...]) + r2bb_ref[...])
    xn_ref[...] = h
    m_ref[...] = h * _dotf(erec_ref[...], rpad_ref[...])


def _edge_out(agg, xji, x, erec, up, r1aw, r1ab, r1bw, r1bb, linw, linb,
              r2aw, r2ab, r2bw, r2bb, rpad):
    full = lambda s: pl.BlockSpec(s, lambda i: (0, 0))
    return pl.pallas_call(
        _edge_out_body,
        grid=(E // EBLK,),
        in_specs=[pl.BlockSpec((EBLK, INT_EMB), lambda i: (i, 0)),
                  pl.BlockSpec((EBLK, H), lambda i: (i, 0)),
                  pl.BlockSpec((EBLK, H), lambda i: (i, 0)),
                  pl.BlockSpec((EBLK, 16), lambda i: (i, 0)),
                  full((INT_EMB, H)),
                  full((H, H)), full((1, H)), full((H, H)), full((1, H)),
                  full((H, H)), full((1, H)),
                  full((H, H)), full((1, H)), full((H, H)), full((1, H)),
                  full((16, H))],
        out_specs=[pl.BlockSpec((EBLK, H), lambda i: (i, 0)),
                   pl.BlockSpec((EBLK, H), lambda i: (i, 0))],
        out_shape=[jax.ShapeDtypeStruct((E, H), jnp.float32),
                   jax.ShapeDtypeStruct((E, H), jnp.float32)],
    )(agg, xji, x, erec, up, r1aw, r1ab, r1bw, r1bb, linw, linb,
      r2aw, r2ab, r2bw, r2bb, rpad)


def _out_dense_body(a0_ref, a1_ref, up_ref, l1w_ref, l1b_ref, l2w_ref,
                    l2b_ref, l3w_ref, l3b_ref, ow_ref, p_ref):
    t = _dotf(a0_ref[...] + a1_ref[...], up_ref[...])
    t = _swish(_dotf(t, l1w_ref[...]) + l1b_ref[...])
    t = _swish(_dotf(t, l2w_ref[...]) + l2b_ref[...])
    t = _swish(_dotf(t, l3w_ref[...]) + l3b_ref[...])
    p_ref[...] = _dotf(t, ow_ref[...])


def _out_dense(a0, a1, up, l1w, l1b, l2w, l2b, l3w, l3b, ow):
    full = lambda s: pl.BlockSpec(s, lambda i: (0, 0))
    return pl.pallas_call(
        _out_dense_body,
        grid=(N // NBLK,),
        in_specs=[pl.BlockSpec((NBLK, H), lambda i: (i, 0)),
                  pl.BlockSpec((NBLK, H), lambda i: (i, 0)),
                  full((H, OUT_EMB)),
                  full((OUT_EMB, OUT_EMB)), full((1, OUT_EMB)),
                  full((OUT_EMB, OUT_EMB)), full((1, OUT_EMB)),
                  full((OUT_EMB, OUT_EMB)), full((1, OUT_EMB)),
                  full((OUT_EMB, OUT_C))],
        out_specs=pl.BlockSpec((NBLK, OUT_C), lambda i: (i, 0)),
        out_shape=jax.ShapeDtypeStruct((N, OUT_C), jnp.float32),
    )(a0, a1, up, l1w, l1b, l2w, l2b, l3w, l3b, ow)


def _final_body(p_ref, oh_ref, bng_ref, bnb_ref, w_ref, b_ref, o_ref, g_ref):
    step = pl.program_id(0)

    @pl.when(step == 0)
    def _():
        g_ref[...] = jnp.zeros_like(g_ref)

    g_ref[...] += jax.lax.dot_general(
        oh_ref[...], p_ref[...], (((0,), (0,)), ((), ())),
        precision=jax.lax.Precision.HIGHEST,
        preferred_element_type=jnp.float32)

    @pl.when(step == pl.num_programs(0) - 1)
    def _():
        g = g_ref[...]
        mu = jnp.mean(g, axis=0, keepdims=True)
        var = jnp.mean((g - mu) ** 2, axis=0, keepdims=True)
        y = (g - mu) / jnp.sqrt(var + 1e-5) * bng_ref[...] + bnb_ref[...]
        y = jnp.where(y >= 0, y, 0.01 * y)
        o_ref[...] = _dotf(y, w_ref[...]) + b_ref[...]


def _final(p, oh, bng, bnb, w, b):
    full = lambda s: pl.BlockSpec(s, lambda i: (0, 0))
    return pl.pallas_call(
        _final_body,
        grid=(N // NBLK,),
        in_specs=[pl.BlockSpec((NBLK, OUT_C), lambda i: (i, 0)),
                  pl.BlockSpec((NBLK, B), lambda i: (i, 0)),
                  full((1, OUT_C)), full((1, OUT_C)),
                  full((OUT_C, 8)), full((1, 8))],
        out_specs=full((B, 8)),
        out_shape=jax.ShapeDtypeStruct((B, 8), jnp.float32),
        scratch_shapes=[pltpu.VMEM((B, OUT_C), jnp.float32)],
    )(p, oh, bng, bnb, w, b)


def kernel(z, pos, batch, edge_index, idx_kj, idx_ji, params):
    f32 = jnp.float32
    j = edge_index[0]
    i = edge_index[1]

    vec = pos[i] - pos[j]
    ss = jnp.sum(vec * vec, axis=-1)
    erec0 = jnp.concatenate(
        [vec, ss[:, None], z[j][:, None].astype(f32), z[i][:, None].astype(f32),
         jnp.zeros((E, 2), f32)], axis=1)

    p = params
    w1 = p['emb_lin_w'][0:H]
    w2 = p['emb_lin_w'][H:2 * H]
    w3 = p['emb_lin_w'][2 * H:3 * H]
    tw1 = jnp.zeros((96, H), f32).at[0:95].set(p['emb_table'] @ w1)
    tw2 = jnp.zeros((96, H), f32).at[0:95].set(p['emb_table'] @ w2)
    w3r = jnp.zeros((16, H), f32).at[0:NR].set(p['emb_rbf_w'])
    b3r = p['emb_rbf_b'][None, :]
    blin = p['emb_lin_b'][None, :]

    def rad_pad(w6):
        return jnp.zeros((16, w6.shape[1]), f32).at[4:10].set(SQ2C * w6)

    r0 = jnp.zeros((16, H), f32).at[0:NR].set(p['out'][0]['rbf_w'])

    erec, x, m0 = _edge_init(erec0, tw1, tw2, w3r, b3r, w3, blin, r0)

    v1 = vec[idx_ji]
    v2 = vec[idx_kj]
    d1 = jnp.sqrt(ss + 1e-12)
    cos_a = jnp.sum(v1 * v2, -1) / (d1[idx_ji] * d1[idx_kj] + 1e-9)
    cc = jnp.clip(cos_a, -1.0 + 1e-7, 1.0 - 1e-7)
    chebs = [jnp.ones((T,), f32), cc]
    for _ in range(NS - 2):
        chebs.append(2.0 * cc * chebs[-1] - chebs[-2])
    radT = erec[:, 4:10][idx_kj]
    angrad = jnp.concatenate(
        [jnp.stack(chebs, axis=1), jnp.zeros((T, 1), f32), radT,
         jnp.zeros((T, 2), f32)], axis=1)

    def node_agg(m):
        s = jax.ops.segment_sum(m, i, num_segments=N)
        return s, jnp.zeros_like(s)

    oh = (batch[:, None] == jnp.arange(B)[None, :]).astype(f32)

    a0, a1 = node_agg(m0)
    ob = p['out'][0]
    P = _out_dense(a0, a1, ob['up_w'], ob['l1_w'], ob['l1_b'][None, :],
                   ob['l2_w'], ob['l2_b'][None, :], ob['l3_w'],
                   ob['l3_b'][None, :], ob['out_w'])

    for b in range(NB):
        ip = p['int'][b]
        wpad = jnp.stack([
            jnp.zeros((16, INT_EMB), f32).at[8:14].set(
                (ip['sbf1'] @ ip['sbf2']).reshape(NS, NR, INT_EMB)[l])
            for l in range(NS)])
        semb = _sbf_emb(angrad, wpad)
        xji, xkj = _edge_in(x, erec, ip['ji_w'], ip['ji_b'][None, :],
                            ip['kj_w'], ip['kj_b'][None, :],
                            rad_pad(ip['rbf1'] @ ip['rbf2']), ip['down'])
        m_trip = xkj[idx_kj] * semb
        agg = jax.ops.segment_sum(m_trip, idx_ji, num_segments=E)

        rpad = rad_pad(p['out'][b + 1]['rbf_w'])
        x, m = _edge_out(agg, xji, x, erec, ip['up'],
                         ip['res1a_w'], ip['res1a_b'][None, :],
                         ip['res1b_w'], ip['res1b_b'][None, :],
                         ip['lin_w'], ip['lin_b'][None, :],
                         ip['res2a_w'], ip['res2a_b'][None, :],
                         ip['res2b_w'], ip['res2b_b'][None, :], rpad)
        a0, a1 = node_agg(m)
        ob = p['out'][b + 1]
        P = P + _out_dense(a0, a1, ob['up_w'], ob['l1_w'], ob['l1_b'][None, :],
                           ob['l2_w'], ob['l2_b'][None, :], ob['l3_w'],
                           ob['l3_b'][None, :], ob['out_w'])

    w8 = jnp.zeros((OUT_C, 8), f32).at[:, 0:1].set(p['lin_w'])
    b8 = jnp.zeros((1, 8), f32).at[:, 0:1].set(p['lin_b'][None, :])
    y8 = _final(P, oh, p['bn_g'][None, :], p['bn_b'][None, :], w8, b8)
    return y8[:, 0:1]


# speedup vs baseline: 1.2893x; 1.2893x over previous
import math

import jax
import jax.numpy as jnp
import numpy as np
from jax.experimental import pallas as pl
from jax.experimental.pallas import tpu as pltpu

N = 10000
E = 160000
T = 480000
B = 256
H = 128
NR = 6
NS = 7
INT_EMB = 64
BAS = 8
OUT_EMB = 256
OUT_C = 16
NB = 4
CUTOFF = 5.0

SQ2C = math.sqrt(2.0 / CUTOFF)
EBLK = 2000
NBLK = 2000
TBLK = 4000


def _sig(x):
    return 1.0 / (1.0 + jnp.exp(-x))


def _swish(x):
    return x * _sig(x)


def _dotf(a, b):
    return jax.lax.dot_general(a, b, (((1,), (0,)), ((), ())),
                               precision=jax.lax.Precision.DEFAULT,
                               preferred_element_type=jnp.float32)


def _sel(rows, cols, pairs):
    m = np.zeros((rows, cols), np.float32)
    for r, c in pairs:
        m[r, c] = 1.0
    return jnp.asarray(m)


def _edge_init_body(erec0_ref, sels_ref, selz_ref, fr_ref,
                    tw1_ref, tw2_ref, w3r_ref, b3r_ref, w3_ref,
                    blin_ref, r0_ref, erec_ref, x_ref, m0_ref):
    e0 = erec0_ref[...]
    ss16 = _dotf(e0, sels_ref[0])
    dist16 = jnp.sqrt(ss16 + 1e-12)
    inv16 = 1.0 / dist16
    freq_a = fr_ref[0:1]
    freq_b = fr_ref[1:2]
    mask3 = fr_ref[2:3]
    rbf16 = SQ2C * jnp.sin(dist16 * freq_a) * inv16
    rad16 = jnp.sin(dist16 * freq_b) * inv16
    xyz = _dotf(e0, sels_ref[1])
    erec_ref[...] = xyz + dist16 * mask3 + rad16

    zj96 = _dotf(e0, selz_ref[0])
    zi96 = _dotf(e0, selz_ref[1])
    iota96 = jax.lax.broadcasted_iota(jnp.int32, (1, 96), 1).astype(jnp.float32)
    ohj = (zj96 == iota96).astype(jnp.float32)
    ohi = (zi96 == iota96).astype(jnp.float32)
    rbf_e = _swish(_dotf(rbf16, w3r_ref[...]) + b3r_ref[...])
    x = _swish(_dotf(ohj, tw1_ref[...]) + _dotf(ohi, tw2_ref[...]) +
               _dotf(rbf_e, w3_ref[...]) + blin_ref[...])
    x_ref[...] = x
    m0_ref[...] = x * _dotf(rbf16, r0_ref[...])


def _edge_init(erec0, tw1, tw2, w3r, b3r, w3, blin, r0):
    sels = jnp.stack([_sel(8, 16, [(3, c) for c in range(16)]),
                      _sel(8, 16, [(0, 0), (1, 1), (2, 2)])])
    selz = jnp.stack([_sel(8, 96, [(4, c) for c in range(96)]),
                      _sel(8, 96, [(5, c) for c in range(96)])])
    n_pi_c = np.array([(n + 1) * math.pi / CUTOFF for n in range(NR)],
                      np.float32)
    fr = np.zeros((3, 16), np.float32)
    fr[0, 0:NR] = n_pi_c
    fr[1, 4:4 + NR] = n_pi_c
    fr[2, 3] = 1.0
    full = lambda s: pl.BlockSpec(s, lambda *a: tuple(0 for _ in s))
    return pl.pallas_call(
        _edge_init_body,
        grid=(E // EBLK,),
        in_specs=[pl.BlockSpec((EBLK, 8), lambda i: (i, 0)),
                  full((2, 8, 16)), full((2, 8, 96)), full((3, 16)),
                  full((96, H)), full((96, H)), full((16, H)), full((1, H)),
                  full((H, H)), full((1, H)), full((16, H))],
        out_specs=[pl.BlockSpec((EBLK, 16), lambda i: (i, 0)),
                   pl.BlockSpec((EBLK, H), lambda i: (i, 0)),
                   pl.BlockSpec((EBLK, H), lambda i: (i, 0))],
        out_shape=[jax.ShapeDtypeStruct((E, 16), jnp.float32),
                   jax.ShapeDtypeStruct((E, H), jnp.float32),
                   jax.ShapeDtypeStruct((E, H), jnp.float32)],
    )(erec0, sels, selz, jnp.asarray(fr), tw1, tw2, w3r, b3r, w3, blin, r0)


def _edge_in_body(x_ref, erec_ref, jiw_ref, jib_ref, kjw_ref, kjb_ref,
                  wrbf_ref, down_ref, xji_ref, xkj_ref):
    x = x_ref[...]
    xji_ref[...] = _swish(_dotf(x, jiw_ref[...]) + jib_ref[...])
    t = _swish(_dotf(x, kjw_ref[...]) + kjb_ref[...])
    t = t * _dotf(erec_ref[...], wrbf_ref[...])
    xkj_ref[...] = _swish(_dotf(t, down_ref[...]))


def _edge_in(x, erec, jiw, jib, kjw, kjb, wrbf, down):
    full = lambda s: pl.BlockSpec(s, lambda i: (0, 0))
    return pl.pallas_call(
        _edge_in_body,
        grid=(E // EBLK,),
        in_specs=[pl.BlockSpec((EBLK, H), lambda i: (i, 0)),
                  pl.BlockSpec((EBLK, 16), lambda i: (i, 0)),
                  full((H, H)), full((1, H)), full((H, H)), full((1, H)),
                  full((16, H)), full((H, INT_EMB))],
        out_specs=[pl.BlockSpec((EBLK, H), lambda i: (i, 0)),
                   pl.BlockSpec((EBLK, INT_EMB), lambda i: (i, 0))],
        out_shape=[jax.ShapeDtypeStruct((E, H), jnp.float32),
                   jax.ShapeDtypeStruct((E, INT_EMB), jnp.float32)],
    )(x, erec, jiw, jib, kjw, kjb, wrbf, down)


def _sbf_emb_body(ar_ref, wpad_ref, bsel_ref, semb_ref):
    ar = ar_ref[...]
    acc = jnp.zeros((ar.shape[0], INT_EMB), jnp.float32)
    for l in range(NS):
        acc = acc + _dotf(ar, bsel_ref[l]) * _dotf(ar, wpad_ref[l])
    semb_ref[...] = acc


def _sbf_emb(angrad, wpad):
    bsel = jnp.stack([_sel(16, INT_EMB, [(l, c) for c in range(INT_EMB)])
                      for l in range(NS)])
    return pl.pallas_call(
        _sbf_emb_body,
        grid=(T // TBLK,),
        in_specs=[pl.BlockSpec((TBLK, 16), lambda i: (i, 0)),
                  pl.BlockSpec((NS, 16, INT_EMB), lambda i: (0, 0, 0)),
                  pl.BlockSpec((NS, 16, INT_EMB), lambda i: (0, 0, 0))],
        out_specs=pl.BlockSpec((TBLK, INT_EMB), lambda i: (i, 0)),
        out_shape=jax.ShapeDtypeStruct((T, INT_EMB), jnp.float32),
    )(angrad, wpad, bsel)


def _edge_out_body(agg_ref, xji_ref, x_ref, erec_ref, up_ref,
                   r1aw_ref, r1ab_ref, r1bw_ref, r1bb_ref,
                   linw_ref, linb_ref,
                   r2aw_ref, r2ab_ref, r2bw_ref, r2bb_ref,
                   rpad_ref, xn_ref, m_ref):
    h = xji_ref[...] + _swish(_dotf(agg_ref[...], up_ref[...]))
    h = h + _swish(_dotf(_swish(_dotf(h, r1aw_ref[...]) + r1ab_ref[...]),
                         r1bw_ref[...]) + r1bb_ref[...])
    h = _swish(_dotf(h, linw_ref[...]) + linb_ref[...]) + x_ref[...]
    h = h + _swish(_dotf(_swish(_dotf(h, r2aw_ref[...]) + r2ab_ref[...]),
                         r2bw_ref[...]) + r2bb_ref[...])
    xn_ref[...] = h
    m_ref[...] = h * _dotf(erec_ref[...], rpad_ref[...])


def _edge_out(agg, xji, x, erec, up, r1aw, r1ab, r1bw, r1bb, linw, linb,
              r2aw, r2ab, r2bw, r2bb, rpad):
    full = lambda s: pl.BlockSpec(s, lambda i: (0, 0))
    return pl.pallas_call(
        _edge_out_body,
        grid=(E // EBLK,),
        in_specs=[pl.BlockSpec((EBLK, INT_EMB), lambda i: (i, 0)),
                  pl.BlockSpec((EBLK, H), lambda i: (i, 0)),
                  pl.BlockSpec((EBLK, H), lambda i: (i, 0)),
                  pl.BlockSpec((EBLK, 16), lambda i: (i, 0)),
                  full((INT_EMB, H)),
                  full((H, H)), full((1, H)), full((H, H)), full((1, H)),
                  full((H, H)), full((1, H)),
                  full((H, H)), full((1, H)), full((H, H)), full((1, H)),
                  full((16, H))],
        out_specs=[pl.BlockSpec((EBLK, H), lambda i: (i, 0)),
                   pl.BlockSpec((EBLK, H), lambda i: (i, 0))],
        out_shape=[jax.ShapeDtypeStruct((E, H), jnp.float32),
                   jax.ShapeDtypeStruct((E, H), jnp.float32)],
    )(agg, xji, x, erec, up, r1aw, r1ab, r1bw, r1bb, linw, linb,
      r2aw, r2ab, r2bw, r2bb, rpad)


def _out_dense_body(a0_ref, a1_ref, up_ref, l1w_ref, l1b_ref, l2w_ref,
                    l2b_ref, l3w_ref, l3b_ref, ow_ref, p_ref):
    t = _dotf(a0_ref[...] + a1_ref[...], up_ref[...])
    t = _swish(_dotf(t, l1w_ref[...]) + l1b_ref[...])
    t = _swish(_dotf(t, l2w_ref[...]) + l2b_ref[...])
    t = _swish(_dotf(t, l3w_ref[...]) + l3b_ref[...])
    p_ref[...] = _dotf(t, ow_ref[...])


def _out_dense(a0, a1, up, l1w, l1b, l2w, l2b, l3w, l3b, ow):
    full = lambda s: pl.BlockSpec(s, lambda i: (0, 0))
    return pl.pallas_call(
        _out_dense_body,
        grid=(N // NBLK,),
        in_specs=[pl.BlockSpec((NBLK, H), lambda i: (i, 0)),
                  pl.BlockSpec((NBLK, H), lambda i: (i, 0)),
                  full((H, OUT_EMB)),
                  full((OUT_EMB, OUT_EMB)), full((1, OUT_EMB)),
                  full((OUT_EMB, OUT_EMB)), full((1, OUT_EMB)),
                  full((OUT_EMB, OUT_EMB)), full((1, OUT_EMB)),
                  full((OUT_EMB, OUT_C))],
        out_specs=pl.BlockSpec((NBLK, OUT_C), lambda i: (i, 0)),
        out_shape=jax.ShapeDtypeStruct((N, OUT_C), jnp.float32),
    )(a0, a1, up, l1w, l1b, l2w, l2b, l3w, l3b, ow)


def _final_body(p_ref, oh_ref, bng_ref, bnb_ref, w_ref, b_ref, o_ref, g_ref):
    step = pl.program_id(0)

    @pl.when(step == 0)
    def _():
        g_ref[...] = jnp.zeros_like(g_ref)

    g_ref[...] += jax.lax.dot_general(
        oh_ref[...], p_ref[...], (((0,), (0,)), ((), ())),
        precision=jax.lax.Precision.HIGHEST,
        preferred_element_type=jnp.float32)

    @pl.when(step == pl.num_programs(0) - 1)
    def _():
        g = g_ref[...]
        mu = jnp.mean(g, axis=0, keepdims=True)
        var = jnp.mean((g - mu) ** 2, axis=0, keepdims=True)
        y = (g - mu) / jnp.sqrt(var + 1e-5) * bng_ref[...] + bnb_ref[...]
        y = jnp.where(y >= 0, y, 0.01 * y)
        o_ref[...] = _dotf(y, w_ref[...]) + b_ref[...]


def _final(p, oh, bng, bnb, w, b):
    full = lambda s: pl.BlockSpec(s, lambda i: (0, 0))
    return pl.pallas_call(
        _final_body,
        grid=(N // NBLK,),
        in_specs=[pl.BlockSpec((NBLK, OUT_C), lambda i: (i, 0)),
                  pl.BlockSpec((NBLK, B), lambda i: (i, 0)),
                  full((1, OUT_C)), full((1, OUT_C)),
                  full((OUT_C, 8)), full((1, 8))],
        out_specs=full((B, 8)),
        out_shape=jax.ShapeDtypeStruct((B, 8), jnp.float32),
        scratch_shapes=[pltpu.VMEM((B, OUT_C), jnp.float32)],
    )(p, oh, bng, bnb, w, b)


def kernel(z, pos, batch, edge_index, idx_kj, idx_ji, params):
    f32 = jnp.float32
    j = edge_index[0]
    i = edge_index[1]

    vec = pos[i] - pos[j]
    ss = jnp.sum(vec * vec, axis=-1)
    erec0 = jnp.concatenate(
        [vec, ss[:, None], z[j][:, None].astype(f32), z[i][:, None].astype(f32),
         jnp.zeros((E, 2), f32)], axis=1)

    p = params
    w1 = p['emb_lin_w'][0:H]
    w2 = p['emb_lin_w'][H:2 * H]
    w3 = p['emb_lin_w'][2 * H:3 * H]
    tw1 = jnp.zeros((96, H), f32).at[0:95].set(p['emb_table'] @ w1)
    tw2 = jnp.zeros((96, H), f32).at[0:95].set(p['emb_table'] @ w2)
    w3r = jnp.zeros((16, H), f32).at[0:NR].set(p['emb_rbf_w'])
    b3r = p['emb_rbf_b'][None, :]
    blin = p['emb_lin_b'][None, :]

    def rad_pad(w6):
        return jnp.zeros((16, w6.shape[1]), f32).at[4:10].set(SQ2C * w6)

    r0 = jnp.zeros((16, H), f32).at[0:NR].set(p['out'][0]['rbf_w'])

    erec, x, m0 = _edge_init(erec0, tw1, tw2, w3r, b3r, w3, blin, r0)

    v1 = vec[idx_ji]
    v2 = vec[idx_kj]
    d1 = jnp.sqrt(ss + 1e-12)
    cos_a = jnp.sum(v1 * v2, -1) / (d1[idx_ji] * d1[idx_kj] + 1e-9)
    cc = jnp.clip(cos_a, -1.0 + 1e-7, 1.0 - 1e-7)
    chebs = [jnp.ones((T,), f32), cc]
    for _ in range(NS - 2):
        chebs.append(2.0 * cc * chebs[-1] - chebs[-2])
    radT = erec[:, 4:10][idx_kj]
    angrad = jnp.concatenate(
        [jnp.stack(chebs, axis=1), jnp.zeros((T, 1), f32), radT,
         jnp.zeros((T, 2), f32)], axis=1)

    def node_agg(m):
        s = jax.ops.segment_sum(m, i, num_segments=N)
        return s, jnp.zeros_like(s)

    oh = (batch[:, None] == jnp.arange(B)[None, :]).astype(f32)

    a0, a1 = node_agg(m0)
    ob = p['out'][0]
    P = _out_dense(a0, a1, ob['up_w'], ob['l1_w'], ob['l1_b'][None, :],
                   ob['l2_w'], ob['l2_b'][None, :], ob['l3_w'],
                   ob['l3_b'][None, :], ob['out_w'])

    for b in range(NB):
        ip = p['int'][b]
        wpad = jnp.stack([
            jnp.zeros((16, INT_EMB), f32).at[8:14].set(
                (ip['sbf1'] @ ip['sbf2']).reshape(NS, NR, INT_EMB)[l])
            for l in range(NS)])
        semb = _sbf_emb(angrad, wpad)
        xji, xkj = _edge_in(x, erec, ip['ji_w'], ip['ji_b'][None, :],
                            ip['kj_w'], ip['kj_b'][None, :],
                            rad_pad(ip['rbf1'] @ ip['rbf2']), ip['down'])
        m_trip = xkj[idx_kj] * semb
        agg = jax.ops.segment_sum(m_trip, idx_ji, num_segments=E)

        rpad = rad_pad(p['out'][b + 1]['rbf_w'])
        x, m = _edge_out(agg, xji, x, erec, ip['up'],
                         ip['res1a_w'], ip['res1a_b'][None, :],
                         ip['res1b_w'], ip['res1b_b'][None, :],
                         ip['lin_w'], ip['lin_b'][None, :],
                         ip['res2a_w'], ip['res2a_b'][None, :],
                         ip['res2b_w'], ip['res2b_b'][None, :], rpad)
        a0, a1 = node_agg(m)
        ob = p['out'][b + 1]
        P = P + _out_dense(a0, a1, ob['up_w'], ob['l1_w'], ob['l1_b'][None, :],
                           ob['l2_w'], ob['l2_b'][None, :], ob['l3_w'],
                           ob['l3_b'][None, :], ob['out_w'])

    w8 = jnp.zeros((OUT_C, 8), f32).at[:, 0:1].set(p['lin_w'])
    b8 = jnp.zeros((1, 8), f32).at[:, 0:1].set(p['lin_b'][None, :])
    y8 = _final(P, oh, p['bn_g'][None, :], p['bn_b'][None, :], w8, b8)
    return y8[:, 0:1]


# speedup vs baseline: 1.5843x; 1.2287x over previous
import functools
import math

import jax
import jax.numpy as jnp
import numpy as np
from jax import lax
from jax.experimental import pallas as pl
from jax.experimental.pallas import tpu as pltpu
from jax.experimental.pallas import tpu_sc as plsc

N = 10000
E = 160000
T = 480000
B = 256
H = 128
NR = 6
NS = 7
INT_EMB = 64
BAS = 8
OUT_EMB = 256
OUT_C = 16
NB = 4
CUTOFF = 5.0

SQ2C = math.sqrt(2.0 / CUTOFF)
EBLK = 2000
NBLK = 2000
TBLK = 4000
SC_NW = 32
GBLK = 120


def _sc_gather(table, idx, width):
    rows_w = T // SC_NW
    nchunk = rows_w // GBLK
    mesh = plsc.VectorSubcoreMesh(core_axis_name="c", subcore_axis_name="s")

    @functools.partial(
        pl.kernel, mesh=mesh,
        out_type=jax.ShapeDtypeStruct((T, width), jnp.float32),
        scratch_types=[pltpu.VMEM((GBLK,), jnp.int32),
                       pltpu.VMEM((GBLK, width), jnp.float32),
                       pltpu.SemaphoreType.DMA],
    )
    def k(table_hbm, idx_hbm, out_hbm, idx_v, rows_v, sem):
        wid = lax.axis_index("s") * 2 + lax.axis_index("c")
        base = wid * rows_w

        def body(c, _):
            off = base + c * GBLK
            pltpu.sync_copy(idx_hbm.at[pl.ds(off, GBLK)], idx_v)
            pltpu.async_copy(table_hbm.at[idx_v], rows_v, sem).wait()
            pltpu.sync_copy(rows_v, out_hbm.at[pl.ds(off, GBLK)])
            return 0

        lax.fori_loop(0, nchunk, body, 0)

    return k(table, idx)


def _sig(x):
    return 1.0 / (1.0 + jnp.exp(-x))


def _swish(x):
    return x * _sig(x)


def _dotf(a, b):
    return jax.lax.dot_general(a, b, (((1,), (0,)), ((), ())),
                               precision=jax.lax.Precision.DEFAULT,
                               preferred_element_type=jnp.float32)


def _sel(rows, cols, pairs):
    m = np.zeros((rows, cols), np.float32)
    for r, c in pairs:
        m[r, c] = 1.0
    return jnp.asarray(m)


def _edge_init_body(erec0_ref, sels_ref, selz_ref, fr_ref,
                    tw1_ref, tw2_ref, w3r_ref, b3r_ref, w3_ref,
                    blin_ref, r0_ref, erec_ref, x_ref, m0_ref):
    e0 = erec0_ref[...]
    ss16 = _dotf(e0, sels_ref[0])
    dist16 = jnp.sqrt(ss16 + 1e-12)
    inv16 = 1.0 / dist16
    freq_a = fr_ref[0:1]
    freq_b = fr_ref[1:2]
    mask3 = fr_ref[2:3]
    rbf16 = SQ2C * jnp.sin(dist16 * freq_a) * inv16
    rad16 = jnp.sin(dist16 * freq_b) * inv16
    xyz = _dotf(e0, sels_ref[1])
    erec_ref[...] = xyz + dist16 * mask3 + rad16

    zj96 = _dotf(e0, selz_ref[0])
    zi96 = _dotf(e0, selz_ref[1])
    iota96 = jax.lax.broadcasted_iota(jnp.int32, (1, 96), 1).astype(jnp.float32)
    ohj = (zj96 == iota96).astype(jnp.float32)
    ohi = (zi96 == iota96).astype(jnp.float32)
    rbf_e = _swish(_dotf(rbf16, w3r_ref[...]) + b3r_ref[...])
    x = _swish(_dotf(ohj, tw1_ref[...]) + _dotf(ohi, tw2_ref[...]) +
               _dotf(rbf_e, w3_ref[...]) + blin_ref[...])
    x_ref[...] = x
    m0_ref[...] = x * _dotf(rbf16, r0_ref[...])


def _edge_init(erec0, tw1, tw2, w3r, b3r, w3, blin, r0):
    sels = jnp.stack([_sel(8, 16, [(3, c) for c in range(16)]),
                      _sel(8, 16, [(0, 0), (1, 1), (2, 2)])])
    selz = jnp.stack([_sel(8, 96, [(4, c) for c in range(96)]),
                      _sel(8, 96, [(5, c) for c in range(96)])])
    n_pi_c = np.array([(n + 1) * math.pi / CUTOFF for n in range(NR)],
                      np.float32)
    fr = np.zeros((3, 16), np.float32)
    fr[0, 0:NR] = n_pi_c
    fr[1, 4:4 + NR] = n_pi_c
    fr[2, 3] = 1.0
    full = lambda s: pl.BlockSpec(s, lambda *a: tuple(0 for _ in s))
    return pl.pallas_call(
        _edge_init_body,
        grid=(E // EBLK,),
        in_specs=[pl.BlockSpec((EBLK, 8), lambda i: (i, 0)),
                  full((2, 8, 16)), full((2, 8, 96)), full((3, 16)),
                  full((96, H)), full((96, H)), full((16, H)), full((1, H)),
                  full((H, H)), full((1, H)), full((16, H))],
        out_specs=[pl.BlockSpec((EBLK, 16), lambda i: (i, 0)),
                   pl.BlockSpec((EBLK, H), lambda i: (i, 0)),
                   pl.BlockSpec((EBLK, H), lambda i: (i, 0))],
        out_shape=[jax.ShapeDtypeStruct((E, 16), jnp.float32),
                   jax.ShapeDtypeStruct((E, H), jnp.float32),
                   jax.ShapeDtypeStruct((E, H), jnp.float32)],
    )(erec0, sels, selz, jnp.asarray(fr), tw1, tw2, w3r, b3r, w3, blin, r0)


def _edge_in_body(x_ref, erec_ref, jiw_ref, jib_ref, kjw_ref, kjb_ref,
                  wrbf_ref, down_ref, xji_ref, xkj_ref):
    x = x_ref[...]
    xji_ref[...] = _swish(_dotf(x, jiw_ref[...]) + jib_ref[...])
    t = _swish(_dotf(x, kjw_ref[...]) + kjb_ref[...])
    t = t * _dotf(erec_ref[...], wrbf_ref[...])
    xkj_ref[...] = _swish(_dotf(t, down_ref[...]))


def _edge_in(x, erec, jiw, jib, kjw, kjb, wrbf, down):
    full = lambda s: pl.BlockSpec(s, lambda i: (0, 0))
    return pl.pallas_call(
        _edge_in_body,
        grid=(E // EBLK,),
        in_specs=[pl.BlockSpec((EBLK, H), lambda i: (i, 0)),
                  pl.BlockSpec((EBLK, 16), lambda i: (i, 0)),
                  full((H, H)), full((1, H)), full((H, H)), full((1, H)),
                  full((16, H)), full((H, H))],
        out_specs=[pl.BlockSpec((EBLK, H), lambda i: (i, 0)),
                   pl.BlockSpec((EBLK, H), lambda i: (i, 0))],
        out_shape=[jax.ShapeDtypeStruct((E, H), jnp.float32),
                   jax.ShapeDtypeStruct((E, H), jnp.float32)],
    )(x, erec, jiw, jib, kjw, kjb, wrbf, down)


def _sbf_emb_body(ar_ref, g_ref, wpad_ref, bsel_ref, semb_ref):
    ar = ar_ref[...]
    acc = jnp.zeros((ar.shape[0], INT_EMB), jnp.float32)
    for l in range(NS):
        acc = acc + _dotf(ar, bsel_ref[l]) * _dotf(ar, wpad_ref[l])
    semb_ref[...] = acc * g_ref[:, 0:INT_EMB]


def _sbf_emb(angrad, gathered, wpad):
    bsel = jnp.stack([_sel(16, INT_EMB, [(l, c) for c in range(INT_EMB)])
                      for l in range(NS)])
    return pl.pallas_call(
        _sbf_emb_body,
        grid=(T // TBLK,),
        in_specs=[pl.BlockSpec((TBLK, 16), lambda i: (i, 0)),
                  pl.BlockSpec((TBLK, H), lambda i: (i, 0)),
                  pl.BlockSpec((NS, 16, INT_EMB), lambda i: (0, 0, 0)),
                  pl.BlockSpec((NS, 16, INT_EMB), lambda i: (0, 0, 0))],
        out_specs=pl.BlockSpec((TBLK, INT_EMB), lambda i: (i, 0)),
        out_shape=jax.ShapeDtypeStruct((T, INT_EMB), jnp.float32),
    )(angrad, gathered, wpad, bsel)


def _edge_out_body(agg_ref, xji_ref, x_ref, erec_ref, up_ref,
                   r1aw_ref, r1ab_ref, r1bw_ref, r1bb_ref,
                   linw_ref, linb_ref,
                   r2aw_ref, r2ab_ref, r2bw_ref, r2bb_ref,
                   rpad_ref, xn_ref, m_ref):
    h = xji_ref[...] + _swish(_dotf(agg_ref[...], up_ref[...]))
    h = h + _swish(_dotf(_swish(_dotf(h, r1aw_ref[...]) + r1ab_ref[...]),
                         r1bw_ref[...]) + r1bb_ref[...])
    h = _swish(_dotf(h, linw_ref[...]) + linb_ref[...]) + x_ref[...]
    h = h + _swish(_dotf(_swish(_dotf(h, r2aw_ref[...]) + r2ab_ref[...]),
                         r2bw_ref[...]) + r2bb_ref[...])
    xn_ref[...] = h
    m_ref[...] = h * _dotf(erec_ref[...], rpad_ref[...])


def _edge_out(agg, xji, x, erec, up, r1aw, r1ab, r1bw, r1bb, linw, linb,
              r2aw, r2ab, r2bw, r2bb, rpad):
    full = lambda s: pl.BlockSpec(s, lambda i: (0, 0))
    return pl.pallas_call(
        _edge_out_body,
        grid=(E // EBLK,),
        in_specs=[pl.BlockSpec((EBLK, INT_EMB), lambda i: (i, 0)),
                  pl.BlockSpec((EBLK, H), lambda i: (i, 0)),
                  pl.BlockSpec((EBLK, H), lambda i: (i, 0)),
                  pl.BlockSpec((EBLK, 16), lambda i: (i, 0)),
                  full((INT_EMB, H)),
                  full((H, H)), full((1, H)), full((H, H)), full((1, H)),
                  full((H, H)), full((1, H)),
                  full((H, H)), full((1, H)), full((H, H)), full((1, H)),
                  full((16, H))],
        out_specs=[pl.BlockSpec((EBLK, H), lambda i: (i, 0)),
                   pl.BlockSpec((EBLK, H), lambda i: (i, 0))],
        out_shape=[jax.ShapeDtypeStruct((E, H), jnp.float32),
                   jax.ShapeDtypeStruct((E, H), jnp.float32)],
    )(agg, xji, x, erec, up, r1aw, r1ab, r1bw, r1bb, linw, linb,
      r2aw, r2ab, r2bw, r2bb, rpad)


def _out_dense_body(a0_ref, a1_ref, up_ref, l1w_ref, l1b_ref, l2w_ref,
                    l2b_ref, l3w_ref, l3b_ref, ow_ref, p_ref):
    t = _dotf(a0_ref[...] + a1_ref[...], up_ref[...])
    t = _swish(_dotf(t, l1w_ref[...]) + l1b_ref[...])
    t = _swish(_dotf(t, l2w_ref[...]) + l2b_ref[...])
    t = _swish(_dotf(t, l3w_ref[...]) + l3b_ref[...])
    p_ref[...] = _dotf(t, ow_ref[...])


def _out_dense(a0, a1, up, l1w, l1b, l2w, l2b, l3w, l3b, ow):
    full = lambda s: pl.BlockSpec(s, lambda i: (0, 0))
    return pl.pallas_call(
        _out_dense_body,
        grid=(N // NBLK,),
        in_specs=[pl.BlockSpec((NBLK, H), lambda i: (i, 0)),
                  pl.BlockSpec((NBLK, H), lambda i: (i, 0)),
                  full((H, OUT_EMB)),
                  full((OUT_EMB, OUT_EMB)), full((1, OUT_EMB)),
                  full((OUT_EMB, OUT_EMB)), full((1, OUT_EMB)),
                  full((OUT_EMB, OUT_EMB)), full((1, OUT_EMB)),
                  full((OUT_EMB, OUT_C))],
        out_specs=pl.BlockSpec((NBLK, OUT_C), lambda i: (i, 0)),
        out_shape=jax.ShapeDtypeStruct((N, OUT_C), jnp.float32),
    )(a0, a1, up, l1w, l1b, l2w, l2b, l3w, l3b, ow)


def _final_body(p_ref, oh_ref, bng_ref, bnb_ref, w_ref, b_ref, o_ref, g_ref):
    step = pl.program_id(0)

    @pl.when(step == 0)
    def _():
        g_ref[...] = jnp.zeros_like(g_ref)

    g_ref[...] += jax.lax.dot_general(
        oh_ref[...], p_ref[...], (((0,), (0,)), ((), ())),
        precision=jax.lax.Precision.HIGHEST,
        preferred_element_type=jnp.float32)

    @pl.when(step == pl.num_programs(0) - 1)
    def _():
        g = g_ref[...]
        mu = jnp.mean(g, axis=0, keepdims=True)
        var = jnp.mean((g - mu) ** 2, axis=0, keepdims=True)
        y = (g - mu) / jnp.sqrt(var + 1e-5) * bng_ref[...] + bnb_ref[...]
        y = jnp.where(y >= 0, y, 0.01 * y)
        o_ref[...] = _dotf(y, w_ref[...]) + b_ref[...]


def _final(p, oh, bng, bnb, w, b):
    full = lambda s: pl.BlockSpec(s, lambda i: (0, 0))
    return pl.pallas_call(
        _final_body,
        grid=(N // NBLK,),
        in_specs=[pl.BlockSpec((NBLK, OUT_C), lambda i: (i, 0)),
                  pl.BlockSpec((NBLK, B), lambda i: (i, 0)),
                  full((1, OUT_C)), full((1, OUT_C)),
                  full((OUT_C, 8)), full((1, 8))],
        out_specs=full((B, 8)),
        out_shape=jax.ShapeDtypeStruct((B, 8), jnp.float32),
        scratch_shapes=[pltpu.VMEM((B, OUT_C), jnp.float32)],
    )(p, oh, bng, bnb, w, b)


def kernel(z, pos, batch, edge_index, idx_kj, idx_ji, params):
    f32 = jnp.float32
    j = edge_index[0]
    i = edge_index[1]

    vec = pos[i] - pos[j]
    ss = jnp.sum(vec * vec, axis=-1)
    erec0 = jnp.concatenate(
        [vec, ss[:, None], z[j][:, None].astype(f32), z[i][:, None].astype(f32),
         jnp.zeros((E, 2), f32)], axis=1)

    p = params
    w1 = p['emb_lin_w'][0:H]
    w2 = p['emb_lin_w'][H:2 * H]
    w3 = p['emb_lin_w'][2 * H:3 * H]
    tw1 = jnp.zeros((96, H), f32).at[0:95].set(p['emb_table'] @ w1)
    tw2 = jnp.zeros((96, H), f32).at[0:95].set(p['emb_table'] @ w2)
    w3r = jnp.zeros((16, H), f32).at[0:NR].set(p['emb_rbf_w'])
    b3r = p['emb_rbf_b'][None, :]
    blin = p['emb_lin_b'][None, :]

    def rad_pad(w6):
        return jnp.zeros((16, w6.shape[1]), f32).at[4:10].set(SQ2C * w6)

    r0 = jnp.zeros((16, H), f32).at[0:NR].set(p['out'][0]['rbf_w'])

    erec, x, m0 = _edge_init(erec0, tw1, tw2, w3r, b3r, w3, blin, r0)

    v1 = vec[idx_ji]
    v2 = vec[idx_kj]
    d1 = jnp.sqrt(ss + 1e-12)
    cos_a = jnp.sum(v1 * v2, -1) / (d1[idx_ji] * d1[idx_kj] + 1e-9)
    cc = jnp.clip(cos_a, -1.0 + 1e-7, 1.0 - 1e-7)
    chebs = [jnp.ones((T,), f32), cc]
    for _ in range(NS - 2):
        chebs.append(2.0 * cc * chebs[-1] - chebs[-2])
    radT = erec[:, 4:10][idx_kj]
    angrad = jnp.concatenate(
        [jnp.stack(chebs, axis=1), jnp.zeros((T, 1), f32), radT,
         jnp.zeros((T, 2), f32)], axis=1)

    def node_agg(m):
        s = jax.ops.segment_sum(m, i, num_segments=N)
        return s, jnp.zeros_like(s)

    oh = (batch[:, None] == jnp.arange(B)[None, :]).astype(f32)

    a0, a1 = node_agg(m0)
    ob = p['out'][0]
    P = _out_dense(a0, a1, ob['up_w'], ob['l1_w'], ob['l1_b'][None, :],
                   ob['l2_w'], ob['l2_b'][None, :], ob['l3_w'],
                   ob['l3_b'][None, :], ob['out_w'])

    for b in range(NB):
        ip = p['int'][b]
        wpad = jnp.stack([
            jnp.zeros((16, INT_EMB), f32).at[8:14].set(
                (ip['sbf1'] @ ip['sbf2']).reshape(NS, NR, INT_EMB)[l])
            for l in range(NS)])
        down128 = jnp.zeros((H, H), f32).at[:, 0:INT_EMB].set(ip['down'])
        xji, xkj = _edge_in(x, erec, ip['ji_w'], ip['ji_b'][None, :],
                            ip['kj_w'], ip['kj_b'][None, :],
                            rad_pad(ip['rbf1'] @ ip['rbf2']), down128)
        gath = _sc_gather(xkj, idx_kj, H)
        m_trip = _sbf_emb(angrad, gath, wpad)
        agg = jax.ops.segment_sum(m_trip, idx_ji, num_segments=E)

        rpad = rad_pad(p['out'][b + 1]['rbf_w'])
        x, m = _edge_out(agg, xji, x, erec, ip['up'],
                         ip['res1a_w'], ip['res1a_b'][None, :],
                         ip['res1b_w'], ip['res1b_b'][None, :],
                         ip['lin_w'], ip['lin_b'][None, :],
                         ip['res2a_w'], ip['res2a_b'][None, :],
                         ip['res2b_w'], ip['res2b_b'][None, :], rpad)
        a0, a1 = node_agg(m)
        ob = p['out'][b + 1]
        P = P + _out_dense(a0, a1, ob['up_w'], ob['l1_w'], ob['l1_b'][None, :],
                           ob['l2_w'], ob['l2_b'][None, :], ob['l3_w'],
                           ob['l3_b'][None, :], ob['out_w'])

    w8 = jnp.zeros((OUT_C, 8), f32).at[:, 0:1].set(p['lin_w'])
    b8 = jnp.zeros((1, 8), f32).at[:, 0:1].set(p['lin_b'][None, :])
    y8 = _final(P, oh, p['bn_g'][None, :], p['bn_b'][None, :], w8, b8)
    return y8[:, 0:1]


# speedup vs baseline: 1.6090x; 1.0156x over previous
import functools
import math

import jax
import jax.numpy as jnp
import numpy as np
from jax import lax
from jax.experimental import pallas as pl
from jax.experimental.pallas import tpu as pltpu
from jax.experimental.pallas import tpu_sc as plsc

N = 10000
E = 160000
T = 480000
B = 256
H = 128
NR = 6
NS = 7
INT_EMB = 64
BAS = 8
OUT_EMB = 256
OUT_C = 16
NB = 4
CUTOFF = 5.0

SQ2C = math.sqrt(2.0 / CUTOFF)
EBLK = 2000
NBLK = 2000
TBLK = 4000
SC_NW = 32
GBLK = 200


def _sc_gather(table, idx, width):
    rows_w = T // SC_NW
    nchunk = rows_w // GBLK
    nbuf = 3
    nouter = nchunk // nbuf
    mesh = plsc.VectorSubcoreMesh(core_axis_name="c", subcore_axis_name="s")

    @functools.partial(
        pl.kernel, mesh=mesh,
        out_type=jax.ShapeDtypeStruct((T, width), jnp.float32),
        scratch_types=[pltpu.VMEM((rows_w,), jnp.int32),
                       pltpu.VMEM((nbuf, GBLK, width), jnp.float32),
                       pltpu.SemaphoreType.DMA,
                       pltpu.SemaphoreType.DMA,
                       pltpu.SemaphoreType.DMA],
    )
    def k(table_hbm, idx_hbm, out_hbm, idx_v, bufs, s0, s1, s2):
        sems = [s0, s1, s2]
        wid = lax.axis_index("s") * 2 + lax.axis_index("c")
        base = wid * rows_w
        pltpu.sync_copy(idx_hbm.at[pl.ds(base, rows_w)], idx_v)

        def start(c, b):
            pltpu.async_copy(table_hbm.at[idx_v.at[pl.ds(c * GBLK, GBLK)]],
                             bufs.at[b], sems[b])

        def drain(c, b):
            pltpu.make_async_copy(
                table_hbm.at[idx_v.at[pl.ds(c * GBLK, GBLK)]],
                bufs.at[b], sems[b]).wait()

        for b in range(nbuf):
            start(b, b)

        def outer(o, _):
            for b in range(nbuf):
                c = o * nbuf + b
                drain(c, b)
                pltpu.sync_copy(bufs.at[b],
                                out_hbm.at[pl.ds(base + c * GBLK, GBLK)])

                @pl.when(c + nbuf < nchunk)
                def _():
                    start(c + nbuf, b)
            return 0

        lax.fori_loop(0, nouter, outer, 0)

    return k(table, idx)


def _sig(x):
    return 1.0 / (1.0 + jnp.exp(-x))


def _swish(x):
    return x * _sig(x)


def _dotf(a, b):
    return jax.lax.dot_general(a, b, (((1,), (0,)), ((), ())),
                               precision=jax.lax.Precision.DEFAULT,
                               preferred_element_type=jnp.float32)


def _sel(rows, cols, pairs):
    m = np.zeros((rows, cols), np.float32)
    for r, c in pairs:
        m[r, c] = 1.0
    return jnp.asarray(m)


def _edge_init_body(erec0_ref, sels_ref, selz_ref, fr_ref,
                    tw1_ref, tw2_ref, w3r_ref, b3r_ref, w3_ref,
                    blin_ref, r0_ref, erec_ref, x_ref, m0_ref):
    e0 = erec0_ref[...]
    ss16 = _dotf(e0, sels_ref[0])
    dist16 = jnp.sqrt(ss16 + 1e-12)
    inv16 = 1.0 / dist16
    freq_a = fr_ref[0:1]
    freq_b = fr_ref[1:2]
    mask3 = fr_ref[2:3]
    rbf16 = SQ2C * jnp.sin(dist16 * freq_a) * inv16
    rad16 = jnp.sin(dist16 * freq_b) * inv16
    xyz = _dotf(e0, sels_ref[1])
    erec_ref[...] = xyz + dist16 * mask3 + rad16

    zj96 = _dotf(e0, selz_ref[0])
    zi96 = _dotf(e0, selz_ref[1])
    iota96 = jax.lax.broadcasted_iota(jnp.int32, (1, 96), 1).astype(jnp.float32)
    ohj = (zj96 == iota96).astype(jnp.float32)
    ohi = (zi96 == iota96).astype(jnp.float32)
    rbf_e = _swish(_dotf(rbf16, w3r_ref[...]) + b3r_ref[...])
    x = _swish(_dotf(ohj, tw1_ref[...]) + _dotf(ohi, tw2_ref[...]) +
               _dotf(rbf_e, w3_ref[...]) + blin_ref[...])
    x_ref[...] = x
    m0_ref[...] = x * _dotf(rbf16, r0_ref[...])


def _edge_init(erec0, tw1, tw2, w3r, b3r, w3, blin, r0):
    sels = jnp.stack([_sel(8, 16, [(3, c) for c in range(16)]),
                      _sel(8, 16, [(0, 0), (1, 1), (2, 2)])])
    selz = jnp.stack([_sel(8, 96, [(4, c) for c in range(96)]),
                      _sel(8, 96, [(5, c) for c in range(96)])])
    n_pi_c = np.array([(n + 1) * math.pi / CUTOFF for n in range(NR)],
                      np.float32)
    fr = np.zeros((3, 16), np.float32)
    fr[0, 0:NR] = n_pi_c
    fr[1, 4:4 + NR] = n_pi_c
    fr[2, 3] = 1.0
    full = lambda s: pl.BlockSpec(s, lambda *a: tuple(0 for _ in s))
    return pl.pallas_call(
        _edge_init_body,
        grid=(E // EBLK,),
        in_specs=[pl.BlockSpec((EBLK, 8), lambda i: (i, 0)),
                  full((2, 8, 16)), full((2, 8, 96)), full((3, 16)),
                  full((96, H)), full((96, H)), full((16, H)), full((1, H)),
                  full((H, H)), full((1, H)), full((16, H))],
        out_specs=[pl.BlockSpec((EBLK, 16), lambda i: (i, 0)),
                   pl.BlockSpec((EBLK, H), lambda i: (i, 0)),
                   pl.BlockSpec((EBLK, H), lambda i: (i, 0))],
        out_shape=[jax.ShapeDtypeStruct((E, 16), jnp.float32),
                   jax.ShapeDtypeStruct((E, H), jnp.float32),
                   jax.ShapeDtypeStruct((E, H), jnp.float32)],
    )(erec0, sels, selz, jnp.asarray(fr), tw1, tw2, w3r, b3r, w3, blin, r0)


def _edge_in_body(x_ref, erec_ref, jiw_ref, jib_ref, kjw_ref, kjb_ref,
                  wrbf_ref, down_ref, xji_ref, xkj_ref):
    x = x_ref[...]
    xji_ref[...] = _swish(_dotf(x, jiw_ref[...]) + jib_ref[...])
    t = _swish(_dotf(x, kjw_ref[...]) + kjb_ref[...])
    t = t * _dotf(erec_ref[...], wrbf_ref[...])
    xkj_ref[...] = _swish(_dotf(t, down_ref[...]))


def _edge_in(x, erec, jiw, jib, kjw, kjb, wrbf, down):
    full = lambda s: pl.BlockSpec(s, lambda i: (0, 0))
    return pl.pallas_call(
        _edge_in_body,
        grid=(E // EBLK,),
        in_specs=[pl.BlockSpec((EBLK, H), lambda i: (i, 0)),
                  pl.BlockSpec((EBLK, 16), lambda i: (i, 0)),
                  full((H, H)), full((1, H)), full((H, H)), full((1, H)),
                  full((16, H)), full((H, H))],
        out_specs=[pl.BlockSpec((EBLK, H), lambda i: (i, 0)),
                   pl.BlockSpec((EBLK, H), lambda i: (i, 0))],
        out_shape=[jax.ShapeDtypeStruct((E, H), jnp.float32),
                   jax.ShapeDtypeStruct((E, H), jnp.float32)],
    )(x, erec, jiw, jib, kjw, kjb, wrbf, down)


def _sbf_emb_body(ar_ref, g_ref, wpad_ref, bsel_ref, semb_ref):
    ar = ar_ref[...]
    acc = jnp.zeros((ar.shape[0], INT_EMB), jnp.float32)
    for l in range(NS):
        acc = acc + _dotf(ar, bsel_ref[l]) * _dotf(ar, wpad_ref[l])
    semb_ref[...] = acc * g_ref[:, 0:INT_EMB]


def _sbf_emb(angrad, gathered, wpad):
    bsel = jnp.stack([_sel(16, INT_EMB, [(l, c) for c in range(INT_EMB)])
                      for l in range(NS)])
    return pl.pallas_call(
        _sbf_emb_body,
        grid=(T // TBLK,),
        in_specs=[pl.BlockSpec((TBLK, 16), lambda i: (i, 0)),
                  pl.BlockSpec((TBLK, H), lambda i: (i, 0)),
                  pl.BlockSpec((NS, 16, INT_EMB), lambda i: (0, 0, 0)),
                  pl.BlockSpec((NS, 16, INT_EMB), lambda i: (0, 0, 0))],
        out_specs=pl.BlockSpec((TBLK, INT_EMB), lambda i: (i, 0)),
        out_shape=jax.ShapeDtypeStruct((T, INT_EMB), jnp.float32),
    )(angrad, gathered, wpad, bsel)


def _edge_out_body(agg_ref, xji_ref, x_ref, erec_ref, up_ref,
                   r1aw_ref, r1ab_ref, r1bw_ref, r1bb_ref,
                   linw_ref, linb_ref,
                   r2aw_ref, r2ab_ref, r2bw_ref, r2bb_ref,
                   rpad_ref, xn_ref, m_ref):
    h = xji_ref[...] + _swish(_dotf(agg_ref[...], up_ref[...]))
    h = h + _swish(_dotf(_swish(_dotf(h, r1aw_ref[...]) + r1ab_ref[...]),
                         r1bw_ref[...]) + r1bb_ref[...])
    h = _swish(_dotf(h, linw_ref[...]) + linb_ref[...]) + x_ref[...]
    h = h + _swish(_dotf(_swish(_dotf(h, r2aw_ref[...]) + r2ab_ref[...]),
                         r2bw_ref[...]) + r2bb_ref[...])
    xn_ref[...] = h
    m_ref[...] = h * _dotf(erec_ref[...], rpad_ref[...])


def _edge_out(agg, xji, x, erec, up, r1aw, r1ab, r1bw, r1bb, linw, linb,
              r2aw, r2ab, r2bw, r2bb, rpad):
    full = lambda s: pl.BlockSpec(s, lambda i: (0, 0))
    return pl.pallas_call(
        _edge_out_body,
        grid=(E // EBLK,),
        in_specs=[pl.BlockSpec((EBLK, INT_EMB), lambda i: (i, 0)),
                  pl.BlockSpec((EBLK, H), lambda i: (i, 0)),
                  pl.BlockSpec((EBLK, H), lambda i: (i, 0)),
                  pl.BlockSpec((EBLK, 16), lambda i: (i, 0)),
                  full((INT_EMB, H)),
                  full((H, H)), full((1, H)), full((H, H)), full((1, H)),
                  full((H, H)), full((1, H)),
                  full((H, H)), full((1, H)), full((H, H)), full((1, H)),
                  full((16, H))],
        out_specs=[pl.BlockSpec((EBLK, H), lambda i: (i, 0)),
                   pl.BlockSpec((EBLK, H), lambda i: (i, 0))],
        out_shape=[jax.ShapeDtypeStruct((E, H), jnp.float32),
                   jax.ShapeDtypeStruct((E, H), jnp.float32)],
    )(agg, xji, x, erec, up, r1aw, r1ab, r1bw, r1bb, linw, linb,
      r2aw, r2ab, r2bw, r2bb, rpad)


def _out_dense_body(a0_ref, a1_ref, up_ref, l1w_ref, l1b_ref, l2w_ref,
                    l2b_ref, l3w_ref, l3b_ref, ow_ref, p_ref):
    t = _dotf(a0_ref[...] + a1_ref[...], up_ref[...])
    t = _swish(_dotf(t, l1w_ref[...]) + l1b_ref[...])
    t = _swish(_dotf(t, l2w_ref[...]) + l2b_ref[...])
    t = _swish(_dotf(t, l3w_ref[...]) + l3b_ref[...])
    p_ref[...] = _dotf(t, ow_ref[...])


def _out_dense(a0, a1, up, l1w, l1b, l2w, l2b, l3w, l3b, ow):
    full = lambda s: pl.BlockSpec(s, lambda i: (0, 0))
    return pl.pallas_call(
        _out_dense_body,
        grid=(N // NBLK,),
        in_specs=[pl.BlockSpec((NBLK, H), lambda i: (i, 0)),
                  pl.BlockSpec((NBLK, H), lambda i: (i, 0)),
                  full((H, OUT_EMB)),
                  full((OUT_EMB, OUT_EMB)), full((1, OUT_EMB)),
                  full((OUT_EMB, OUT_EMB)), full((1, OUT_EMB)),
                  full((OUT_EMB, OUT_EMB)), full((1, OUT_EMB)),
                  full((OUT_EMB, OUT_C))],
        out_specs=pl.BlockSpec((NBLK, OUT_C), lambda i: (i, 0)),
        out_shape=jax.ShapeDtypeStruct((N, OUT_C), jnp.float32),
    )(a0, a1, up, l1w, l1b, l2w, l2b, l3w, l3b, ow)


def _final_body(p_ref, oh_ref, bng_ref, bnb_ref, w_ref, b_ref, o_ref, g_ref):
    step = pl.program_id(0)

    @pl.when(step == 0)
    def _():
        g_ref[...] = jnp.zeros_like(g_ref)

    g_ref[...] += jax.lax.dot_general(
        oh_ref[...], p_ref[...], (((0,), (0,)), ((), ())),
        precision=jax.lax.Precision.HIGHEST,
        preferred_element_type=jnp.float32)

    @pl.when(step == pl.num_programs(0) - 1)
    def _():
        g = g_ref[...]
        mu = jnp.mean(g, axis=0, keepdims=True)
        var = jnp.mean((g - mu) ** 2, axis=0, keepdims=True)
        y = (g - mu) / jnp.sqrt(var + 1e-5) * bng_ref[...] + bnb_ref[...]
        y = jnp.where(y >= 0, y, 0.01 * y)
        o_ref[...] = _dotf(y, w_ref[...]) + b_ref[...]


def _final(p, oh, bng, bnb, w, b):
    full = lambda s: pl.BlockSpec(s, lambda i: (0, 0))
    return pl.pallas_call(
        _final_body,
        grid=(N // NBLK,),
        in_specs=[pl.BlockSpec((NBLK, OUT_C), lambda i: (i, 0)),
                  pl.BlockSpec((NBLK, B), lambda i: (i, 0)),
                  full((1, OUT_C)), full((1, OUT_C)),
                  full((OUT_C, 8)), full((1, 8))],
        out_specs=full((B, 8)),
        out_shape=jax.ShapeDtypeStruct((B, 8), jnp.float32),
        scratch_shapes=[pltpu.VMEM((B, OUT_C), jnp.float32)],
    )(p, oh, bng, bnb, w, b)


def kernel(z, pos, batch, edge_index, idx_kj, idx_ji, params):
    f32 = jnp.float32
    j = edge_index[0]
    i = edge_index[1]

    vec = pos[i] - pos[j]
    ss = jnp.sum(vec * vec, axis=-1)
    erec0 = jnp.concatenate(
        [vec, ss[:, None], z[j][:, None].astype(f32), z[i][:, None].astype(f32),
         jnp.zeros((E, 2), f32)], axis=1)

    p = params
    w1 = p['emb_lin_w'][0:H]
    w2 = p['emb_lin_w'][H:2 * H]
    w3 = p['emb_lin_w'][2 * H:3 * H]
    tw1 = jnp.zeros((96, H), f32).at[0:95].set(p['emb_table'] @ w1)
    tw2 = jnp.zeros((96, H), f32).at[0:95].set(p['emb_table'] @ w2)
    w3r = jnp.zeros((16, H), f32).at[0:NR].set(p['emb_rbf_w'])
    b3r = p['emb_rbf_b'][None, :]
    blin = p['emb_lin_b'][None, :]

    def rad_pad(w6):
        return jnp.zeros((16, w6.shape[1]), f32).at[4:10].set(SQ2C * w6)

    r0 = jnp.zeros((16, H), f32).at[0:NR].set(p['out'][0]['rbf_w'])

    erec, x, m0 = _edge_init(erec0, tw1, tw2, w3r, b3r, w3, blin, r0)

    v1 = vec[idx_ji]
    v2 = vec[idx_kj]
    d1 = jnp.sqrt(ss + 1e-12)
    cos_a = jnp.sum(v1 * v2, -1) / (d1[idx_ji] * d1[idx_kj] + 1e-9)
    cc = jnp.clip(cos_a, -1.0 + 1e-7, 1.0 - 1e-7)
    chebs = [jnp.ones((T,), f32), cc]
    for _ in range(NS - 2):
        chebs.append(2.0 * cc * chebs[-1] - chebs[-2])
    radT = erec[:, 4:10][idx_kj]
    angrad = jnp.concatenate(
        [jnp.stack(chebs, axis=1), jnp.zeros((T, 1), f32), radT,
         jnp.zeros((T, 2), f32)], axis=1)

    def node_agg(m):
        s = jax.ops.segment_sum(m, i, num_segments=N)
        return s, jnp.zeros_like(s)

    oh = (batch[:, None] == jnp.arange(B)[None, :]).astype(f32)

    a0, a1 = node_agg(m0)
    ob = p['out'][0]
    P = _out_dense(a0, a1, ob['up_w'], ob['l1_w'], ob['l1_b'][None, :],
                   ob['l2_w'], ob['l2_b'][None, :], ob['l3_w'],
                   ob['l3_b'][None, :], ob['out_w'])

    for b in range(NB):
        ip = p['int'][b]
        wpad = jnp.stack([
            jnp.zeros((16, INT_EMB), f32).at[8:14].set(
                (ip['sbf1'] @ ip['sbf2']).reshape(NS, NR, INT_EMB)[l])
            for l in range(NS)])
        down128 = jnp.zeros((H, H), f32).at[:, 0:INT_EMB].set(ip['down'])
        xji, xkj = _edge_in(x, erec, ip['ji_w'], ip['ji_b'][None, :],
                            ip['kj_w'], ip['kj_b'][None, :],
                            rad_pad(ip['rbf1'] @ ip['rbf2']), down128)
        gath = _sc_gather(xkj, idx_kj, H)
        m_trip = _sbf_emb(angrad, gath, wpad)
        agg = jax.ops.segment_sum(m_trip, idx_ji, num_segments=E)

        rpad = rad_pad(p['out'][b + 1]['rbf_w'])
        x, m = _edge_out(agg, xji, x, erec, ip['up'],
                         ip['res1a_w'], ip['res1a_b'][None, :],
                         ip['res1b_w'], ip['res1b_b'][None, :],
                         ip['lin_w'], ip['lin_b'][None, :],
                         ip['res2a_w'], ip['res2a_b'][None, :],
                         ip['res2b_w'], ip['res2b_b'][None, :], rpad)
        a0, a1 = node_agg(m)
        ob = p['out'][b + 1]
        P = P + _out_dense(a0, a1, ob['up_w'], ob['l1_w'], ob['l1_b'][None, :],
                           ob['l2_w'], ob['l2_b'][None, :], ob['l3_w'],
                           ob['l3_b'][None, :], ob['out_w'])

    w8 = jnp.zeros((OUT_C, 8), f32).at[:, 0:1].set(p['lin_w'])
    b8 = jnp.zeros((1, 8), f32).at[:, 0:1].set(p['lin_b'][None, :])
    y8 = _final(P, oh, p['bn_g'][None, :], p['bn_b'][None, :], w8, b8)
    return y8[:, 0:1]
